# baseline (speedup 1.0000x reference)
import functools

import jax
import jax.numpy as jnp
from jax.experimental import pallas as pl
from jax.experimental.pallas import tpu as pltpu

_VMEM_LIMIT = 60 * 1024 * 1024

_C1N = 1408
_C2N = 384
_KFC = 33640
_KP = 36864


def _conv1_pool_kernel(x_ref, bw_ref, bias_ref, o_ref):
    bb = x_ref.shape[0]
    xs = jnp.concatenate([x_ref[:, s:s + 62, :] for s in range(3)], axis=2)
    y = jnp.dot(xs.reshape(bb * 62, 768), bw_ref[...],
                preferred_element_type=jnp.float32)
    y = jnp.maximum(y + bias_ref[...], 0.0)
    y = jnp.maximum(y[:, :2 * _C1N], y[:, 2 * _C1N:])
    y = jnp.maximum(y[:, :_C1N], y[:, _C1N:])
    o_ref[...] = y.reshape(bb, 62, _C1N)


def _conv2_pool_kernel(p_ref, bw_ref, bias_ref, o_ref):
    bb = p_ref.shape[0]
    zs = []
    for t in range(4):
        xt = jnp.concatenate(
            [p_ref[:, s:s + 29, 1408 * p + 320 * t:1408 * p + 320 * t + 400]
             for s in range(3) for p in range(2)], axis=2)
        yt = jnp.dot(xt.reshape(bb * 29, 2400), bw_ref[...],
                     preferred_element_type=jnp.float32)
        yt = jnp.maximum(yt + bias_ref[...], 0.0)
        yt = jnp.maximum(yt[:, :2 * _C2N], yt[:, 2 * _C2N:])
        yt = jnp.maximum(yt[:, :_C2N], yt[:, _C2N:])
        zs.append(yt[:, :320])
    z = jnp.concatenate(zs, axis=1)
    o_ref[...] = z.reshape(bb, 29, 1280)


def _fc_kernel(x_ref, w1_ref, b1_ref, w2_ref, b2_ref, o_ref, acc_ref):
    k = pl.program_id(0)

    @pl.when(k == 0)
    def _():
        acc_ref[...] = jnp.zeros_like(acc_ref)

    acc_ref[...] += jnp.dot(x_ref[:, :1160], w1_ref[...],
                            preferred_element_type=jnp.float32)

    @pl.when(k == pl.num_programs(0) - 1)
    def _():
        h = jnp.maximum(acc_ref[...] + b1_ref[...], 0.0)
        logits = jnp.dot(h, w2_ref[...],
                         preferred_element_type=jnp.float32) + b2_ref[...]
        m = jnp.max(logits, axis=1, keepdims=True)
        s = logits - m
        lse = jnp.log(jnp.sum(jnp.exp(s), axis=1, keepdims=True))
        o_ref[...] = (s - lse).astype(o_ref.dtype)


def _banded_weights(conv1_w, conv1_b, conv2_w, conv2_b):
    f32 = jnp.float32
    r = (2 * jnp.arange(3)[:, None] + jnp.arange(2)[None, :]).reshape(6)
    a = [(r[:, None] - hp == jnp.arange(5)[None, :]).astype(f32)
         for hp in range(2)]
    w1 = conv1_w.reshape(5, 5, 20)
    blocks, biases = [], []
    bias1 = jnp.concatenate(
        [jnp.tile(conv1_b, (1, 62)), jnp.zeros((1, _C1N - 1240), f32)], axis=1)
    for hp in range(2):
        for wp in range(2):
            c = (jnp.arange(128)[:, None, None]
                 == 2 * jnp.arange(68)[None, :, None] + wp
                 + jnp.arange(5)[None, None, :]).astype(f32)
            blk = jnp.einsum('si,wuj,ijc->swuc', a[hp], c, w1)
            blk = blk.reshape(768, 68 * 20)
            blk = jnp.concatenate(
                [blk, jnp.zeros((768, _C1N - 1360), f32)], axis=1)
            blocks.append(blk)
            biases.append(bias1)
    bw1 = jnp.concatenate(blocks, axis=1)
    bias1_full = jnp.concatenate(biases, axis=1)
    w2 = conv2_w.reshape(5, 5, 20, 40)
    blocks2, biases2 = [], []
    bias2 = jnp.concatenate(
        [jnp.tile(conv2_b, (1, 8)), jnp.zeros((1, _C2N - 320), f32)], axis=1)
    for hp in range(2):
        for wp in range(2):
            c = (jnp.arange(20)[:, None, None]
                 == 2 * jnp.arange(8)[None, :, None] + wp
                 + jnp.arange(5)[None, None, :]).astype(f32)
            blk = jnp.einsum('si,kuj,ijcd->skcud', a[hp], c, w2)
            blk = blk.reshape(2400, 320)
            blk = jnp.concatenate(
                [blk, jnp.zeros((2400, _C2N - 320), f32)], axis=1)
            blocks2.append(blk)
            biases2.append(bias2)
    bw2 = jnp.concatenate(blocks2, axis=1)
    bias2_full = jnp.concatenate(biases2, axis=1)
    return bw1, bias1_full, bw2, bias2_full


def kernel(x, conv1_w, conv1_b, conv2_w, conv2_b, fc1_w, fc1_b, fc2_w, fc2_b):
    n = x.shape[0]
    xs = x.reshape(n, 64, 256)
    bw1, bias1, bw2, bias2 = _banded_weights(conv1_w, conv1_b,
                                             conv2_w, conv2_b)

    bb = 8
    p1 = pl.pallas_call(
        _conv1_pool_kernel,
        out_shape=jax.ShapeDtypeStruct((n, 62, _C1N), jnp.float32),
        grid_spec=pltpu.PrefetchScalarGridSpec(
            num_scalar_prefetch=0,
            grid=(n // bb,),
            in_specs=[
                pl.BlockSpec((bb, 64, 256), lambda i: (i, 0, 0)),
                pl.BlockSpec((768, 4 * _C1N), lambda i: (0, 0)),
                pl.BlockSpec((1, 4 * _C1N), lambda i: (0, 0)),
            ],
            out_specs=pl.BlockSpec((bb, 62, _C1N), lambda i: (i, 0, 0)),
        ),
        compiler_params=pltpu.CompilerParams(
            dimension_semantics=("parallel",),
            vmem_limit_bytes=_VMEM_LIMIT),
    )(xs, bw1, bias1)

    p1v = p1.reshape(n, 31, 2 * _C1N)

    p2 = pl.pallas_call(
        _conv2_pool_kernel,
        out_shape=jax.ShapeDtypeStruct((n, 29, 1280), jnp.float32),
        grid_spec=pltpu.PrefetchScalarGridSpec(
            num_scalar_prefetch=0,
            grid=(n // bb,),
            in_specs=[
                pl.BlockSpec((bb, 31, 2 * _C1N), lambda i: (i, 0, 0)),
                pl.BlockSpec((2400, 4 * _C2N), lambda i: (0, 0)),
                pl.BlockSpec((1, 4 * _C2N), lambda i: (0, 0)),
            ],
            out_specs=pl.BlockSpec((bb, 29, 1280), lambda i: (i, 0, 0)),
        ),
        compiler_params=pltpu.CompilerParams(
            dimension_semantics=("parallel",),
            vmem_limit_bytes=_VMEM_LIMIT),
    )(p1v, bw2, bias2)

    flat = p2.reshape(n, 29 * 1280)
    return p2
    out = pl.pallas_call(
        _fc_kernel,
        out_shape=jax.ShapeDtypeStruct((n, 6), jnp.float32),
        grid_spec=pltpu.PrefetchScalarGridSpec(
            num_scalar_prefetch=0,
            grid=(29,),
            in_specs=[
                pl.BlockSpec((n, 1280), lambda k: (0, k)),
                pl.BlockSpec((1160, 256), lambda k: (k, 0)),
                pl.BlockSpec((1, 256), lambda k: (0, 0)),
                pl.BlockSpec((256, 6), lambda k: (0, 0)),
                pl.BlockSpec((1, 6), lambda k: (0, 0)),
            ],
            out_specs=pl.BlockSpec((n, 6), lambda k: (0, 0)),
            scratch_shapes=[pltpu.VMEM((n, 256), jnp.float32)],
        ),
        compiler_params=pltpu.CompilerParams(
            dimension_semantics=("arbitrary",),
            vmem_limit_bytes=_VMEM_LIMIT),
    )(flat, fc1_w, fc1_b, fc2_w, fc2_b)
    return out


# speedup vs baseline: 1.0531x; 1.0531x over previous
import functools

import jax
import jax.numpy as jnp
from jax.experimental import pallas as pl
from jax.experimental.pallas import tpu as pltpu

_VMEM_LIMIT = 60 * 1024 * 1024

_C1N = 1360
_C1NP = 1408
_C2N = 384


def _conv1_pool_kernel(x_ref, bw00, bw01, bw10, bw11, bias_ref, o_ref):
    bb = x_ref.shape[0]
    xs = jnp.concatenate([x_ref[:, s:s + 62, :] for s in range(3)], axis=2)
    xs = xs.reshape(bb * 62, 768).astype(jnp.bfloat16)
    ys = [jnp.dot(xs, bw[...], preferred_element_type=jnp.float32)
          for bw in (bw00, bw01, bw10, bw11)]
    y = jnp.maximum(jnp.maximum(ys[0], ys[1]), jnp.maximum(ys[2], ys[3]))
    y = jnp.maximum(y + bias_ref[...], 0.0)
    y = jnp.concatenate(
        [y, jnp.zeros((bb * 62, _C1NP - _C1N), y.dtype)], axis=1)
    o_ref[...] = y.reshape(bb, 62, _C1NP)


def _conv2_pool_kernel(p_ref, bw_ref, bias_ref, o_ref):
    bb = p_ref.shape[0]
    zs = []
    for t in range(4):
        xt = jnp.concatenate(
            [p_ref[:, s:s + 29,
                   _C1NP * p + 320 * t:_C1NP * p + 320 * t + 400]
             for s in range(3) for p in range(2)], axis=2)
        xt = xt.reshape(bb * 29, 2400).astype(jnp.bfloat16)
        yt = jnp.dot(xt, bw_ref[...], preferred_element_type=jnp.float32)
        yt = jnp.maximum(yt + bias_ref[...], 0.0)
        yt = jnp.maximum(yt[:, :2 * _C2N], yt[:, 2 * _C2N:])
        yt = jnp.maximum(yt[:, :_C2N], yt[:, _C2N:])
        zs.append(yt[:, :320])
    z = jnp.concatenate(zs, axis=1)
    o_ref[...] = z.reshape(bb, 29, 1280)


def _fc_kernel(x_ref, w1_ref, b1_ref, w2_ref, b2_ref, o_ref, acc_ref):
    k = pl.program_id(0)

    @pl.when(k == 0)
    def _():
        acc_ref[...] = jnp.zeros_like(acc_ref)

    acc_ref[...] += jnp.dot(x_ref[:, :1160], w1_ref[...],
                            preferred_element_type=jnp.float32)

    @pl.when(k == pl.num_programs(0) - 1)
    def _():
        h = jnp.maximum(acc_ref[...] + b1_ref[...], 0.0)
        logits = jnp.dot(h, w2_ref[...],
                         preferred_element_type=jnp.float32) + b2_ref[...]
        m = jnp.max(logits, axis=1, keepdims=True)
        s = logits - m
        lse = jnp.log(jnp.sum(jnp.exp(s), axis=1, keepdims=True))
        o_ref[...] = (s - lse).astype(o_ref.dtype)


def _banded_weights(conv1_w, conv1_b, conv2_w, conv2_b):
    f32, bf16 = jnp.float32, jnp.bfloat16
    r = (2 * jnp.arange(3)[:, None] + jnp.arange(2)[None, :]).reshape(6)
    a = [(r[:, None] - hp == jnp.arange(5)[None, :]).astype(f32)
         for hp in range(2)]
    w1 = conv1_w.reshape(5, 5, 20)
    bw1 = []
    for hp in range(2):
        for wp in range(2):
            c = (jnp.arange(128)[:, None, None]
                 == 2 * jnp.arange(68)[None, :, None] + wp
                 + jnp.arange(5)[None, None, :]).astype(f32)
            blk = jnp.einsum('si,wuj,ijc->swuc', a[hp], c, w1)
            bw1.append(blk.reshape(768, _C1N).astype(bf16))
    bias1 = jnp.concatenate(
        [jnp.tile(conv1_b, (1, 62)), jnp.zeros((1, _C1N - 1240), f32)],
        axis=1)
    w2 = conv2_w.reshape(5, 5, 20, 40)
    blocks2 = []
    bias2 = jnp.concatenate(
        [jnp.tile(conv2_b, (1, 8)), jnp.zeros((1, _C2N - 320), f32)], axis=1)
    for hp in range(2):
        for wp in range(2):
            c = (jnp.arange(20)[:, None, None]
                 == 2 * jnp.arange(8)[None, :, None] + wp
                 + jnp.arange(5)[None, None, :]).astype(f32)
            blk = jnp.einsum('si,kuj,ijcd->skcud', a[hp], c, w2)
            blk = blk.reshape(2400, 320)
            blocks2.append(jnp.concatenate(
                [blk, jnp.zeros((2400, _C2N - 320), f32)], axis=1))
    bw2 = jnp.concatenate(blocks2, axis=1).astype(bf16)
    bias2_full = jnp.concatenate([bias2] * 4, axis=1)
    return bw1, bias1, bw2, bias2_full


def kernel(x, conv1_w, conv1_b, conv2_w, conv2_b, fc1_w, fc1_b, fc2_w, fc2_b):
    n = x.shape[0]
    xs = x.reshape(n, 64, 256)
    bw1, bias1, bw2, bias2 = _banded_weights(conv1_w, conv1_b,
                                             conv2_w, conv2_b)

    bb = 8
    wspec = pl.BlockSpec((768, _C1N), lambda i: (0, 0))
    p1 = pl.pallas_call(
        _conv1_pool_kernel,
        out_shape=jax.ShapeDtypeStruct((n, 62, _C1NP), jnp.float32),
        grid_spec=pltpu.PrefetchScalarGridSpec(
            num_scalar_prefetch=0,
            grid=(n // bb,),
            in_specs=[
                pl.BlockSpec((bb, 64, 256), lambda i: (i, 0, 0)),
                wspec, wspec, wspec, wspec,
                pl.BlockSpec((1, _C1N), lambda i: (0, 0)),
            ],
            out_specs=pl.BlockSpec((bb, 62, _C1NP), lambda i: (i, 0, 0)),
        ),
        compiler_params=pltpu.CompilerParams(
            dimension_semantics=("parallel",),
            vmem_limit_bytes=_VMEM_LIMIT),
    )(xs, *bw1, bias1)

    p1v = p1.reshape(n, 31, 2 * _C1NP)

    p2 = pl.pallas_call(
        _conv2_pool_kernel,
        out_shape=jax.ShapeDtypeStruct((n, 29, 1280), jnp.float32),
        grid_spec=pltpu.PrefetchScalarGridSpec(
            num_scalar_prefetch=0,
            grid=(n // bb,),
            in_specs=[
                pl.BlockSpec((bb, 31, 2 * _C1NP), lambda i: (i, 0, 0)),
                pl.BlockSpec((2400, 4 * _C2N), lambda i: (0, 0)),
                pl.BlockSpec((1, 4 * _C2N), lambda i: (0, 0)),
            ],
            out_specs=pl.BlockSpec((bb, 29, 1280), lambda i: (i, 0, 0)),
        ),
        compiler_params=pltpu.CompilerParams(
            dimension_semantics=("parallel",),
            vmem_limit_bytes=_VMEM_LIMIT),
    )(p1v, bw2, bias2)

    flat = p2.reshape(n, 29 * 1280)
    out = pl.pallas_call(
        _fc_kernel,
        out_shape=jax.ShapeDtypeStruct((n, 6), jnp.float32),
        grid_spec=pltpu.PrefetchScalarGridSpec(
            num_scalar_prefetch=0,
            grid=(29,),
            in_specs=[
                pl.BlockSpec((n, 1280), lambda k: (0, k)),
                pl.BlockSpec((1160, 256), lambda k: (k, 0)),
                pl.BlockSpec((1, 256), lambda k: (0, 0)),
                pl.BlockSpec((256, 6), lambda k: (0, 0)),
                pl.BlockSpec((1, 6), lambda k: (0, 0)),
            ],
            out_specs=pl.BlockSpec((n, 6), lambda k: (0, 0)),
            scratch_shapes=[pltpu.VMEM((n, 256), jnp.float32)],
        ),
        compiler_params=pltpu.CompilerParams(
            dimension_semantics=("arbitrary",),
            vmem_limit_bytes=_VMEM_LIMIT),
    )(flat, fc1_w, fc1_b, fc2_w, fc2_b)
    return out


# speedup vs baseline: 1.0711x; 1.0170x over previous
import functools

import jax
import jax.numpy as jnp
from jax.experimental import pallas as pl
from jax.experimental.pallas import tpu as pltpu

_VMEM_LIMIT = 60 * 1024 * 1024

_C1N = 1360
_C1NP = 1408
_C2N = 384


def _conv1_pool_kernel(x_ref, bw00, bw01, bw10, bw11, bias_ref, o_ref):
    bb = x_ref.shape[0]
    xs = jnp.concatenate([x_ref[:, s:s + 62, :] for s in range(3)], axis=2)
    xs = xs.reshape(bb * 62, 768).astype(jnp.bfloat16)
    ys = [jnp.dot(xs, bw[...], preferred_element_type=jnp.float32)
          for bw in (bw00, bw01, bw10, bw11)]
    y = jnp.maximum(jnp.maximum(ys[0], ys[1]), jnp.maximum(ys[2], ys[3]))
    y = jnp.maximum(y + bias_ref[...], 0.0)
    y = jnp.concatenate(
        [y, jnp.zeros((bb * 62, _C1NP - _C1N), y.dtype)], axis=1)
    o_ref[...] = y.reshape(bb, 62, _C1NP)


def _conv2_pool_kernel(p_ref, bw_ref, bias_ref, o_ref):
    bb = p_ref.shape[0]
    zs = []
    for t in range(4):
        xt = jnp.concatenate(
            [p_ref[:, s:s + 29,
                   _C1NP * p + 320 * t:_C1NP * p + 320 * t + 400]
             for s in range(3) for p in range(2)], axis=2)
        xt = xt.reshape(bb * 29, 2400).astype(jnp.bfloat16)
        yt = jnp.dot(xt, bw_ref[...], preferred_element_type=jnp.float32)
        yt = jnp.maximum(yt + bias_ref[...], 0.0)
        yt = jnp.maximum(yt[:, :2 * _C2N], yt[:, 2 * _C2N:])
        yt = jnp.maximum(yt[:, :_C2N], yt[:, _C2N:])
        zs.append(yt[:, :320])
    z = jnp.concatenate(zs, axis=1)
    o_ref[...] = z.reshape(bb, 29, 1280)


def _fc_kernel(x_ref, w1_ref, b1_ref, w2_ref, b2_ref, o_ref, acc_ref):
    k = pl.program_id(0)

    @pl.when(k == 0)
    def _():
        acc_ref[...] = jnp.zeros_like(acc_ref)

    acc_ref[...] += jnp.dot(x_ref[:, :1160], w1_ref[...],
                            preferred_element_type=jnp.float32)

    @pl.when(k == pl.num_programs(0) - 1)
    def _():
        h = jnp.maximum(acc_ref[...] + b1_ref[...], 0.0)
        logits = jnp.dot(h, w2_ref[...],
                         preferred_element_type=jnp.float32) + b2_ref[...]
        m = jnp.max(logits, axis=1, keepdims=True)
        s = logits - m
        lse = jnp.log(jnp.sum(jnp.exp(s), axis=1, keepdims=True))
        o_ref[...] = (s - lse).astype(o_ref.dtype)


def _banded_weights(conv1_w, conv1_b, conv2_w, conv2_b):
    f32, bf16 = jnp.float32, jnp.bfloat16
    r = (2 * jnp.arange(3)[:, None] + jnp.arange(2)[None, :]).reshape(6)
    a = [(r[:, None] - hp == jnp.arange(5)[None, :]).astype(f32)
         for hp in range(2)]
    w1 = conv1_w.reshape(5, 5, 20)
    bw1 = []
    for hp in range(2):
        for wp in range(2):
            c = (jnp.arange(128)[:, None, None]
                 == 2 * jnp.arange(68)[None, :, None] + wp
                 + jnp.arange(5)[None, None, :]).astype(f32)
            blk = jnp.einsum('si,wuj,ijc->swuc', a[hp], c, w1)
            bw1.append(blk.reshape(768, _C1N).astype(bf16))
    bias1 = jnp.concatenate(
        [jnp.tile(conv1_b, (1, 62)), jnp.zeros((1, _C1N - 1240), f32)],
        axis=1)
    w2 = conv2_w.reshape(5, 5, 20, 40)
    blocks2 = []
    bias2 = jnp.concatenate(
        [jnp.tile(conv2_b, (1, 8)), jnp.zeros((1, _C2N - 320), f32)], axis=1)
    for hp in range(2):
        for wp in range(2):
            c = (jnp.arange(20)[:, None, None]
                 == 2 * jnp.arange(8)[None, :, None] + wp
                 + jnp.arange(5)[None, None, :]).astype(f32)
            blk = jnp.einsum('si,kuj,ijcd->skcud', a[hp], c, w2)
            blk = blk.reshape(2400, 320)
            blocks2.append(jnp.concatenate(
                [blk, jnp.zeros((2400, _C2N - 320), f32)], axis=1))
    bw2 = jnp.concatenate(blocks2, axis=1).astype(bf16)
    bias2_full = jnp.concatenate([bias2] * 4, axis=1)
    return bw1, bias1, bw2, bias2_full


def kernel(x, conv1_w, conv1_b, conv2_w, conv2_b, fc1_w, fc1_b, fc2_w, fc2_b):
    n = x.shape[0]
    xs = x.reshape(n, 64, 256)
    bw1, bias1, bw2, bias2 = _banded_weights(conv1_w, conv1_b,
                                             conv2_w, conv2_b)

    bb = 16
    wspec = pl.BlockSpec((768, _C1N), lambda i: (0, 0))
    p1 = pl.pallas_call(
        _conv1_pool_kernel,
        out_shape=jax.ShapeDtypeStruct((n, 62, _C1NP), jnp.float32),
        grid_spec=pltpu.PrefetchScalarGridSpec(
            num_scalar_prefetch=0,
            grid=(n // bb,),
            in_specs=[
                pl.BlockSpec((bb, 64, 256), lambda i: (i, 0, 0)),
                wspec, wspec, wspec, wspec,
                pl.BlockSpec((1, _C1N), lambda i: (0, 0)),
            ],
            out_specs=pl.BlockSpec((bb, 62, _C1NP), lambda i: (i, 0, 0)),
        ),
        compiler_params=pltpu.CompilerParams(
            dimension_semantics=("parallel",),
            vmem_limit_bytes=_VMEM_LIMIT),
    )(xs, *bw1, bias1)

    p1v = p1.reshape(n, 31, 2 * _C1NP)

    p2 = pl.pallas_call(
        _conv2_pool_kernel,
        out_shape=jax.ShapeDtypeStruct((n, 29, 1280), jnp.float32),
        grid_spec=pltpu.PrefetchScalarGridSpec(
            num_scalar_prefetch=0,
            grid=(n // bb,),
            in_specs=[
                pl.BlockSpec((bb, 31, 2 * _C1NP), lambda i: (i, 0, 0)),
                pl.BlockSpec((2400, 4 * _C2N), lambda i: (0, 0)),
                pl.BlockSpec((1, 4 * _C2N), lambda i: (0, 0)),
            ],
            out_specs=pl.BlockSpec((bb, 29, 1280), lambda i: (i, 0, 0)),
        ),
        compiler_params=pltpu.CompilerParams(
            dimension_semantics=("parallel",),
            vmem_limit_bytes=_VMEM_LIMIT),
    )(p1v, bw2, bias2)

    flat = p2.reshape(n, 29 * 1280)
    out = pl.pallas_call(
        _fc_kernel,
        out_shape=jax.ShapeDtypeStruct((n, 6), jnp.float32),
        grid_spec=pltpu.PrefetchScalarGridSpec(
            num_scalar_prefetch=0,
            grid=(29,),
            in_specs=[
                pl.BlockSpec((n, 1280), lambda k: (0, k)),
                pl.BlockSpec((1160, 256), lambda k: (k, 0)),
                pl.BlockSpec((1, 256), lambda k: (0, 0)),
                pl.BlockSpec((256, 6), lambda k: (0, 0)),
                pl.BlockSpec((1, 6), lambda k: (0, 0)),
            ],
            out_specs=pl.BlockSpec((n, 6), lambda k: (0, 0)),
            scratch_shapes=[pltpu.VMEM((n, 256), jnp.float32)],
        ),
        compiler_params=pltpu.CompilerParams(
            dimension_semantics=("arbitrary",),
            vmem_limit_bytes=_VMEM_LIMIT),
    )(flat, fc1_w, fc1_b, fc2_w, fc2_b)
    return out
